# baseline (speedup 1.0000x reference)
import functools

import jax
import jax.numpy as jnp
from jax import lax
from jax.experimental import pallas as pl
from jax.experimental.pallas import tpu as pltpu
from jax.experimental.pallas import tpu_sc as plsc

N = 10000
E = 320000
D = 128
NC = 2
NS = 16
NW = NC * NS
EPT = E // NW
KCH = 80
NCHUNK = EPT // KCH
RPT = N // NS
BATCH = 2

_mesh = plsc.VectorSubcoreMesh(core_axis_name="c", subcore_axis_name="s")


HROW = 80
EPAD = HROW * 128


def _deg_body(dst_hbm, out_hbm, dstbuf, hist):
    c = lax.axis_index("c")
    s = lax.axis_index("s")
    wid = s * NC + c
    ones = jnp.ones((16,), jnp.float32)
    zeros = jnp.zeros((16,), jnp.float32)
    for b in range(BATCH):
        row = b * NW + wid
        pltpu.sync_copy(dst_hbm.at[row], dstbuf)

        def zero_body(i, carry):
            hist[lax.div(i, 8), pl.ds(lax.rem(i, 8) * 16, 16)] = zeros
            return carry

        lax.fori_loop(0, HROW * 8, zero_body, 0)

        def hist_body(i, carry):
            idx = dstbuf[lax.div(i, 8), pl.ds(lax.rem(i, 8) * 16, 16)]
            plsc.addupdate_scatter(
                hist,
                [lax.shift_right_logical(idx, 7),
                 lax.bitwise_and(idx, 127)],
                ones,
            )
            return carry

        lax.fori_loop(0, HROW * 8, hist_body, 0)
        pltpu.sync_copy(hist, out_hbm.at[row])


_deg_kernel = pl.kernel(
    _deg_body,
    out_type=jax.ShapeDtypeStruct((BATCH * NW, HROW, 128), jnp.float32),
    mesh=_mesh,
    compiler_params=pltpu.CompilerParams(needs_layout_passes=False),
    scratch_types=[
        pltpu.VMEM((HROW, 128), jnp.int32),
        pltpu.VMEM((HROW, 128), jnp.float32),
    ],
)


EPTP = 10240
KC2 = 128
NCH_H = 40
NHALF = 2
ACC_R = N + 16


def _edge_body(hs_hbm, src_hbm, dst_hbm, out_hbm, idx_src, idx_dst, rows0,
               rows1, acc, sem0, sem1):
    c = lax.axis_index("c")
    s = lax.axis_index("s")
    wid = s * NC + c
    nblk = 7 + jnp.where(s < 13, 1, 0)
    r80 = rows0.at[pl.ds(0, 80)]

    def src_slice(j):
        return hs_hbm.at[idx_src.at[pl.ds(KC2 * j, KC2)]]

    for b in range(BATCH):
        def init_body(k, carry):
            ro = 80 * s + 1280 * k
            pltpu.sync_copy(hs_hbm.at[pl.ds(b * N + ro, 80)], r80)
            pltpu.sync_copy(r80, acc.at[pl.ds(ro, 80)])
            return carry

        lax.fori_loop(0, nblk, init_body, 0)
        plsc.subcore_barrier()

        for h in range(NHALF):
            pltpu.sync_copy(
                src_hbm.at[pl.ds((b * NW + wid) * EPTP + h * (EPTP // 2),
                                 EPTP // 2)], idx_src)
            pltpu.sync_copy(dst_hbm.at[(b * NW + wid) * NHALF + h], idx_dst)

            pltpu.async_copy(src_slice(0), rows0, sem0)

            def chunk_body(i, carry):
                j0 = 2 * i
                pltpu.async_copy(src_slice(j0 + 1), rows1, sem1)
                pltpu.make_async_copy(src_slice(j0), rows0, sem0).wait()
                pltpu.sync_copy(rows0, acc.at[idx_dst.at[j0]], add=True)
                pltpu.async_copy(src_slice(j0 + 2), rows0, sem0)
                pltpu.make_async_copy(src_slice(j0 + 1), rows1, sem1).wait()
                pltpu.sync_copy(rows1, acc.at[idx_dst.at[j0 + 1]], add=True)
                return carry

            lax.fori_loop(0, (NCH_H - 1) // 2, chunk_body, 0)
            pltpu.async_copy(src_slice(NCH_H - 1), rows1, sem1)
            pltpu.make_async_copy(src_slice(NCH_H - 2), rows0, sem0).wait()
            pltpu.sync_copy(rows0, acc.at[idx_dst.at[NCH_H - 2]], add=True)
            pltpu.make_async_copy(src_slice(NCH_H - 1), rows1, sem1).wait()
            pltpu.sync_copy(rows1, acc.at[idx_dst.at[NCH_H - 1]], add=True)

        plsc.subcore_barrier()
        o0 = (b * NC + c) * N

        def out_body(k, carry):
            ro = 80 * s + 1280 * k
            pltpu.sync_copy(acc.at[pl.ds(ro, 80)], r80)
            pltpu.sync_copy(r80, out_hbm.at[pl.ds(o0 + ro, 80)])
            return carry

        lax.fori_loop(0, nblk, out_body, 0)
        plsc.subcore_barrier()


_edge_kernel = pl.kernel(
    _edge_body,
    out_type=jax.ShapeDtypeStruct((BATCH * NC * N, D), jnp.float32),
    mesh=_mesh,
    scratch_types=[
        pltpu.VMEM((EPTP // 2,), jnp.int32),
        pltpu.VMEM((NCH_H, KC2), jnp.int32),
        pltpu.VMEM((KC2, D), jnp.float32),
        pltpu.VMEM((KC2, D), jnp.float32),
        pltpu.VMEM_SHARED((ACC_R, D), jnp.float32),
        pltpu.SemaphoreType.DMA,
        pltpu.SemaphoreType.DMA,
    ],
)


BLK = 2000


def _k1_body(x_ref, w_ref, degp_ref, hs_ref, dinv_ref):
    deg = jnp.sum(degp_ref[0], axis=1) + 1.0
    dinv = lax.rsqrt(deg)[:, None]
    hs = jnp.dot(x_ref[0], w_ref[...], preferred_element_type=jnp.float32)
    hs_ref[0] = hs * dinv
    dinv_ref[0] = dinv


def _tc_k1(x, w1, degp):
    return pl.pallas_call(
        _k1_body,
        grid=(BATCH, N // BLK),
        in_specs=[
            pl.BlockSpec((1, BLK, D), lambda b, i: (b, i, 0)),
            pl.BlockSpec((D, D), lambda b, i: (0, 0)),
            pl.BlockSpec((1, BLK, NW), lambda b, i: (b, i, 0)),
        ],
        out_specs=[
            pl.BlockSpec((1, BLK, D), lambda b, i: (b, i, 0)),
            pl.BlockSpec((1, BLK, 1), lambda b, i: (b, i, 0)),
        ],
        out_shape=[
            jax.ShapeDtypeStruct((BATCH, N, D), jnp.float32),
            jax.ShapeDtypeStruct((BATCH, N, 1), jnp.float32),
        ],
    )(x, w1, degp)


def _k2_body(p_ref, hs_ref, dinv_ref, b1_ref, w2_ref, out_ref):
    dinv = dinv_ref[0]
    agg = p_ref[0, 0] + p_ref[0, 1] - hs_ref[0]
    h1 = dinv * agg + b1_ref[0]
    h1 = jnp.where(h1 > 0, h1, jnp.exp(jnp.minimum(h1, 0.0)) - 1.0)
    hs2 = jnp.dot(h1, w2_ref[...], preferred_element_type=jnp.float32)
    out_ref[0] = hs2 * dinv


def _tc_k2(p, hs, dinv, b1, w2):
    return pl.pallas_call(
        _k2_body,
        grid=(BATCH, N // BLK),
        in_specs=[
            pl.BlockSpec((1, NC, BLK, D), lambda b, i: (b, 0, i, 0)),
            pl.BlockSpec((1, BLK, D), lambda b, i: (b, i, 0)),
            pl.BlockSpec((1, BLK, 1), lambda b, i: (b, i, 0)),
            pl.BlockSpec((1, D), lambda b, i: (0, 0)),
            pl.BlockSpec((D, D), lambda b, i: (0, 0)),
        ],
        out_specs=pl.BlockSpec((1, BLK, D), lambda b, i: (b, i, 0)),
        out_shape=jax.ShapeDtypeStruct((BATCH, N, D), jnp.float32),
    )(p, hs, dinv, b1, w2)


def _k3_body(q_ref, hs_ref, dinv_ref, b2_ref, out_ref):
    dinv = dinv_ref[0]
    agg = q_ref[0, 0] + q_ref[0, 1] - hs_ref[0]
    out_ref[0] = dinv * agg + b2_ref[0]


def _tc_k3(q, hs, dinv, b2):
    return pl.pallas_call(
        _k3_body,
        grid=(BATCH, N // BLK),
        in_specs=[
            pl.BlockSpec((1, NC, BLK, D), lambda b, i: (b, 0, i, 0)),
            pl.BlockSpec((1, BLK, D), lambda b, i: (b, i, 0)),
            pl.BlockSpec((1, BLK, 1), lambda b, i: (b, i, 0)),
            pl.BlockSpec((1, D), lambda b, i: (0, 0)),
        ],
        out_specs=pl.BlockSpec((1, BLK, D), lambda b, i: (b, i, 0)),
        out_shape=jax.ShapeDtypeStruct((BATCH, N, D), jnp.float32),
    )(q, hs, dinv, b2)


@jax.jit
def kernel(x, edge_index, W1, b1, W2, b2):
    src = edge_index[:, 0, :]
    dst = edge_index[:, 1, :]
    shift = (jnp.arange(BATCH, dtype=jnp.int32) * N)[:, None, None]
    src3 = src.reshape(BATCH, NW, EPT) + shift
    src_pad = jnp.concatenate(
        [src3, jnp.broadcast_to(shift, (BATCH, NW, EPTP - EPT))], axis=2)
    src_r = src_pad.reshape(BATCH * NW * EPTP)
    dst3 = dst.reshape(BATCH, NW, EPT)
    dst_pad2 = jnp.concatenate(
        [dst3, jnp.full((BATCH, NW, EPTP - EPT), N, dtype=jnp.int32)],
        axis=2)
    dst_r = dst_pad2.reshape(BATCH * NW * NHALF, NCH_H, KC2)
    dst_pad = jnp.concatenate(
        [dst.reshape(BATCH * NW, EPT),
         jnp.full((BATCH * NW, EPAD - EPT), N, dtype=jnp.int32)], axis=1
    ).reshape(BATCH * NW, HROW, 128)

    degp = (_deg_kernel(dst_pad).reshape(BATCH, NW, EPAD)[:, :, :N]
            .transpose(0, 2, 1))

    hs1, dinv = _tc_k1(x, W1, degp)

    p = _edge_kernel(hs1.reshape(BATCH * N, D), src_r, dst_r)
    p = p.reshape(BATCH, NC, N, D)

    hs2 = _tc_k2(p, hs1, dinv, b1.reshape(1, D), W2)

    q = _edge_kernel(hs2.reshape(BATCH * N, D), src_r, dst_r)
    q = q.reshape(BATCH, NC, N, D)

    return _tc_k3(q, hs2, dinv, b2.reshape(1, D))


# speedup vs baseline: 2.8255x; 2.8255x over previous
import functools

import jax
import jax.numpy as jnp
from jax import lax
from jax.experimental import pallas as pl
from jax.experimental.pallas import tpu as pltpu
from jax.experimental.pallas import tpu_sc as plsc

N = 10000
E = 320000
D = 128
NC = 2
NS = 16
NW = NC * NS
EPT = E // NW
KCH = 80
NCHUNK = EPT // KCH
RPT = N // NS
BATCH = 2

_mesh = plsc.VectorSubcoreMesh(core_axis_name="c", subcore_axis_name="s")


HROW = 80
EPAD = HROW * 128


def _deg_body(dst_hbm, out_hbm, dstbuf, hist):
    c = lax.axis_index("c")
    s = lax.axis_index("s")
    wid = s * NC + c
    ones = jnp.ones((16,), jnp.float32)
    zeros = jnp.zeros((16,), jnp.float32)
    for b in range(BATCH):
        row = b * NW + wid
        pltpu.sync_copy(dst_hbm.at[row], dstbuf)

        def zero_body(i, carry):
            hist[lax.div(i, 8), pl.ds(lax.rem(i, 8) * 16, 16)] = zeros
            return carry

        lax.fori_loop(0, HROW * 8, zero_body, 0)

        def hist_body(i, carry):
            idx = dstbuf[lax.div(i, 8), pl.ds(lax.rem(i, 8) * 16, 16)]
            plsc.addupdate_scatter(
                hist,
                [lax.shift_right_logical(idx, 7),
                 lax.bitwise_and(idx, 127)],
                ones,
            )
            return carry

        lax.fori_loop(0, HROW * 8, hist_body, 0)
        pltpu.sync_copy(hist, out_hbm.at[row])


_deg_kernel = pl.kernel(
    _deg_body,
    out_type=jax.ShapeDtypeStruct((BATCH * NW, HROW, 128), jnp.float32),
    mesh=_mesh,
    compiler_params=pltpu.CompilerParams(needs_layout_passes=False),
    scratch_types=[
        pltpu.VMEM((HROW, 128), jnp.int32),
        pltpu.VMEM((HROW, 128), jnp.float32),
    ],
)


def _edge_body(hs_hbm, src_hbm, dst_hbm, out_hbm, idx_src, idx_dst, rows0,
               rows1, acc, sem0, sem1):
    c = lax.axis_index("c")
    s = lax.axis_index("s")
    wid = s * NC + c
    nblk = 7 + jnp.where(s < 13, 1, 0)

    def src_slice(j):
        return hs_hbm.at[idx_src.at[pl.ds(KCH * j, KCH)]]

    for b in range(BATCH):
        row = b * NW + wid
        pltpu.sync_copy(src_hbm.at[pl.ds(row * EPT, EPT)], idx_src)
        pltpu.sync_copy(dst_hbm.at[row], idx_dst)

        def init_body(k, carry):
            ro = 80 * s + 1280 * k
            pltpu.sync_copy(hs_hbm.at[pl.ds(b * N + ro, 80)], rows0)
            pltpu.sync_copy(rows0, acc.at[pl.ds(ro, 80)])
            return carry

        lax.fori_loop(0, nblk, init_body, 0)
        plsc.subcore_barrier()

        pltpu.async_copy(src_slice(0), rows0, sem0)

        def chunk_body(i, carry):
            j0 = 2 * i
            pltpu.async_copy(src_slice(j0 + 1), rows1, sem1)
            pltpu.make_async_copy(src_slice(j0), rows0, sem0).wait()
            pltpu.sync_copy(rows0, acc.at[idx_dst.at[j0]], add=True)
            pltpu.async_copy(src_slice(j0 + 2), rows0, sem0)
            pltpu.make_async_copy(src_slice(j0 + 1), rows1, sem1).wait()
            pltpu.sync_copy(rows1, acc.at[idx_dst.at[j0 + 1]], add=True)
            return carry

        lax.fori_loop(0, (NCHUNK - 1) // 2, chunk_body, 0)
        pltpu.make_async_copy(src_slice(NCHUNK - 1), rows0, sem0).wait()
        pltpu.sync_copy(rows0, acc.at[idx_dst.at[NCHUNK - 1]], add=True)
        plsc.subcore_barrier()
        o0 = (b * NC + c) * N

        def out_body(k, carry):
            ro = 80 * s + 1280 * k
            pltpu.sync_copy(acc.at[pl.ds(ro, 80)], rows0)
            pltpu.sync_copy(rows0, out_hbm.at[pl.ds(o0 + ro, 80)])
            return carry

        lax.fori_loop(0, nblk, out_body, 0)
        plsc.subcore_barrier()


_edge_kernel = pl.kernel(
    _edge_body,
    out_type=jax.ShapeDtypeStruct((BATCH * NC * N, D), jnp.float32),
    mesh=_mesh,
    scratch_types=[
        pltpu.VMEM((EPT,), jnp.int32),
        pltpu.VMEM((NCHUNK, KCH), jnp.int32),
        pltpu.VMEM((KCH, D), jnp.float32),
        pltpu.VMEM((KCH, D), jnp.float32),
        pltpu.VMEM_SHARED((N, D), jnp.float32),
        pltpu.SemaphoreType.DMA,
        pltpu.SemaphoreType.DMA,
    ],
)


BLK = 2000


def _k1_body(x_ref, w_ref, degp_ref, hs_ref, dinv_ref):
    deg = jnp.sum(degp_ref[0], axis=1) + 1.0
    dinv = lax.rsqrt(deg)[:, None]
    hs = jnp.dot(x_ref[0], w_ref[...], preferred_element_type=jnp.float32)
    hs_ref[0] = hs * dinv
    dinv_ref[0] = dinv


def _tc_k1(x, w1, degp):
    return pl.pallas_call(
        _k1_body,
        grid=(BATCH, N // BLK),
        in_specs=[
            pl.BlockSpec((1, BLK, D), lambda b, i: (b, i, 0)),
            pl.BlockSpec((D, D), lambda b, i: (0, 0)),
            pl.BlockSpec((1, BLK, NW), lambda b, i: (b, i, 0)),
        ],
        out_specs=[
            pl.BlockSpec((1, BLK, D), lambda b, i: (b, i, 0)),
            pl.BlockSpec((1, BLK, 1), lambda b, i: (b, i, 0)),
        ],
        out_shape=[
            jax.ShapeDtypeStruct((BATCH, N, D), jnp.float32),
            jax.ShapeDtypeStruct((BATCH, N, 1), jnp.float32),
        ],
    )(x, w1, degp)


def _k2_body(p_ref, hs_ref, dinv_ref, b1_ref, w2_ref, out_ref):
    dinv = dinv_ref[0]
    agg = p_ref[0, 0] + p_ref[0, 1] - hs_ref[0]
    h1 = dinv * agg + b1_ref[0]
    h1 = jnp.where(h1 > 0, h1, jnp.exp(jnp.minimum(h1, 0.0)) - 1.0)
    hs2 = jnp.dot(h1, w2_ref[...], preferred_element_type=jnp.float32)
    out_ref[0] = hs2 * dinv


def _tc_k2(p, hs, dinv, b1, w2):
    return pl.pallas_call(
        _k2_body,
        grid=(BATCH, N // BLK),
        in_specs=[
            pl.BlockSpec((1, NC, BLK, D), lambda b, i: (b, 0, i, 0)),
            pl.BlockSpec((1, BLK, D), lambda b, i: (b, i, 0)),
            pl.BlockSpec((1, BLK, 1), lambda b, i: (b, i, 0)),
            pl.BlockSpec((1, D), lambda b, i: (0, 0)),
            pl.BlockSpec((D, D), lambda b, i: (0, 0)),
        ],
        out_specs=pl.BlockSpec((1, BLK, D), lambda b, i: (b, i, 0)),
        out_shape=jax.ShapeDtypeStruct((BATCH, N, D), jnp.float32),
    )(p, hs, dinv, b1, w2)


def _k3_body(q_ref, hs_ref, dinv_ref, b2_ref, out_ref):
    dinv = dinv_ref[0]
    agg = q_ref[0, 0] + q_ref[0, 1] - hs_ref[0]
    out_ref[0] = dinv * agg + b2_ref[0]


def _tc_k3(q, hs, dinv, b2):
    return pl.pallas_call(
        _k3_body,
        grid=(BATCH, N // BLK),
        in_specs=[
            pl.BlockSpec((1, NC, BLK, D), lambda b, i: (b, 0, i, 0)),
            pl.BlockSpec((1, BLK, D), lambda b, i: (b, i, 0)),
            pl.BlockSpec((1, BLK, 1), lambda b, i: (b, i, 0)),
            pl.BlockSpec((1, D), lambda b, i: (0, 0)),
        ],
        out_specs=pl.BlockSpec((1, BLK, D), lambda b, i: (b, i, 0)),
        out_shape=jax.ShapeDtypeStruct((BATCH, N, D), jnp.float32),
    )(q, hs, dinv, b2)


@jax.jit
def kernel(x, edge_index, W1, b1, W2, b2):
    src = edge_index[:, 0, :]
    dst = edge_index[:, 1, :]
    src_shift = src + (jnp.arange(BATCH, dtype=jnp.int32) * N)[:, None]
    src_r = src_shift.reshape(BATCH * NW * EPT)
    dst_r = dst.reshape(BATCH * NW, NCHUNK, KCH)
    dst_pad = jnp.concatenate(
        [dst.reshape(BATCH * NW, EPT),
         jnp.full((BATCH * NW, EPAD - EPT), N, dtype=jnp.int32)], axis=1
    ).reshape(BATCH * NW, HROW, 128)

    degp = (_deg_kernel(dst_pad).reshape(BATCH, NW, EPAD)[:, :, :N]
            .transpose(0, 2, 1))

    hs1, dinv = _tc_k1(x, W1, degp)

    p = _edge_kernel(hs1.reshape(BATCH * N, D), src_r, dst_r)
    p = p.reshape(BATCH, NC, N, D)

    hs2 = _tc_k2(p, hs1, dinv, b1.reshape(1, D), W2)

    q = _edge_kernel(hs2.reshape(BATCH * N, D), src_r, dst_r)
    q = q.reshape(BATCH, NC, N, D)

    return _tc_k3(q, hs2, dinv, b2.reshape(1, D))


# speedup vs baseline: 3.0447x; 1.0776x over previous
import functools

import jax
import jax.numpy as jnp
from jax import lax
from jax.experimental import pallas as pl
from jax.experimental.pallas import tpu as pltpu
from jax.experimental.pallas import tpu_sc as plsc

N = 10000
E = 320000
D = 128
NC = 2
NS = 16
NW = NC * NS
EPT = E // NW
KCH = 80
NCHUNK = EPT // KCH
RPT = N // NS
BATCH = 2

_mesh = plsc.VectorSubcoreMesh(core_axis_name="c", subcore_axis_name="s")


HROW = 80
EPAD = HROW * 128


def _deg_body(dst_hbm, out_hbm, dstbuf, hist):
    c = lax.axis_index("c")
    s = lax.axis_index("s")
    wid = s * NC + c
    ones = jnp.ones((16,), jnp.float32)
    zeros = jnp.zeros((16,), jnp.float32)
    for b in range(BATCH):
        row = b * NW + wid
        pltpu.sync_copy(dst_hbm.at[row], dstbuf)

        def zero_body(i, carry):
            hist[lax.div(i, 8), pl.ds(lax.rem(i, 8) * 16, 16)] = zeros
            return carry

        lax.fori_loop(0, HROW * 8, zero_body, 0)

        def hist_body(i, carry):
            idx = dstbuf[lax.div(i, 8), pl.ds(lax.rem(i, 8) * 16, 16)]
            plsc.addupdate_scatter(
                hist,
                [lax.shift_right_logical(idx, 7),
                 lax.bitwise_and(idx, 127)],
                ones,
            )
            return carry

        lax.fori_loop(0, HROW * 8, hist_body, 0)
        pltpu.sync_copy(hist, out_hbm.at[row])


_deg_kernel = pl.kernel(
    _deg_body,
    out_type=jax.ShapeDtypeStruct((BATCH * NW, HROW, 128), jnp.float32),
    mesh=_mesh,
    compiler_params=pltpu.CompilerParams(needs_layout_passes=False),
    scratch_types=[
        pltpu.VMEM((HROW, 128), jnp.int32),
        pltpu.VMEM((HROW, 128), jnp.float32),
    ],
)


def _edge_body(hs_hbm, src_hbm, dst_hbm, out_hbm, idx_src, idx_dst, rows0,
               rows1, acc, sem0, sem1):
    c = lax.axis_index("c")
    s = lax.axis_index("s")
    wid = s * NC + c
    nblk = 7 + jnp.where(s < 13, 1, 0)

    def src_slice(j):
        return hs_hbm.at[idx_src.at[pl.ds(KCH * j, KCH)]]

    zeros16 = jnp.zeros((16,), jnp.float32)
    for b in range(BATCH):
        row = b * NW + wid
        pltpu.async_copy(src_hbm.at[pl.ds(row * EPT, EPT)], idx_src, sem0)
        pltpu.async_copy(dst_hbm.at[row], idx_dst, sem1)

        def zero_body(i, carry):
            rows0[lax.div(i, 8), pl.ds(lax.rem(i, 8) * 16, 16)] = zeros16
            return carry

        lax.fori_loop(0, 80 * 8, zero_body, 0)

        def init_body(k, carry):
            ro = 80 * s + 1280 * k
            pltpu.sync_copy(rows0, acc.at[pl.ds(ro, 80)])
            return carry

        lax.fori_loop(0, nblk, init_body, 0)
        pltpu.make_async_copy(src_hbm.at[pl.ds(row * EPT, EPT)], idx_src,
                              sem0).wait()
        pltpu.make_async_copy(dst_hbm.at[row], idx_dst, sem1).wait()
        plsc.subcore_barrier()

        pltpu.async_copy(src_slice(0), rows0, sem0)

        def chunk_body(i, carry):
            j0 = 2 * i
            pltpu.async_copy(src_slice(j0 + 1), rows1, sem1)
            pltpu.make_async_copy(src_slice(j0), rows0, sem0).wait()
            pltpu.sync_copy(rows0, acc.at[idx_dst.at[j0]], add=True)
            pltpu.async_copy(src_slice(j0 + 2), rows0, sem0)
            pltpu.make_async_copy(src_slice(j0 + 1), rows1, sem1).wait()
            pltpu.sync_copy(rows1, acc.at[idx_dst.at[j0 + 1]], add=True)
            return carry

        lax.fori_loop(0, (NCHUNK - 1) // 2, chunk_body, 0)
        pltpu.make_async_copy(src_slice(NCHUNK - 1), rows0, sem0).wait()
        pltpu.sync_copy(rows0, acc.at[idx_dst.at[NCHUNK - 1]], add=True)
        plsc.subcore_barrier()
        o0 = (b * NC + c) * N

        for k in range(8):
            rbuf = rows0 if k % 2 == 0 else rows1
            rsem = sem0 if k % 2 == 0 else sem1

            @pl.when(k < nblk)
            def _(k=k, rbuf=rbuf, rsem=rsem):
                ro = 80 * s + 1280 * k
                if k >= 2:
                    pltpu.make_async_copy(
                        rbuf, out_hbm.at[pl.ds(o0 + ro - 2560, 80)],
                        rsem).wait()
                pltpu.sync_copy(acc.at[pl.ds(ro, 80)], rbuf)
                pltpu.async_copy(rbuf, out_hbm.at[pl.ds(o0 + ro, 80)],
                                 rsem)

        pltpu.make_async_copy(rows0, out_hbm.at[pl.ds(o0, 80)],
                              sem0).wait()
        pltpu.make_async_copy(rows1, out_hbm.at[pl.ds(o0, 80)],
                              sem1).wait()
        plsc.subcore_barrier()


_edge_kernel = pl.kernel(
    _edge_body,
    out_type=jax.ShapeDtypeStruct((BATCH * NC * N, D), jnp.float32),
    mesh=_mesh,
    compiler_params=pltpu.CompilerParams(needs_layout_passes=False),
    scratch_types=[
        pltpu.VMEM((EPT,), jnp.int32),
        pltpu.VMEM((NCHUNK, KCH), jnp.int32),
        pltpu.VMEM((KCH, D), jnp.float32),
        pltpu.VMEM((KCH, D), jnp.float32),
        pltpu.VMEM_SHARED((N, D), jnp.float32),
        pltpu.SemaphoreType.DMA,
        pltpu.SemaphoreType.DMA,
    ],
)


BLK = 2000


def _k1_body(x_ref, w_ref, degp_ref, hs_ref, dinv_ref):
    deg = jnp.sum(degp_ref[0], axis=1) + 1.0
    dinv = lax.rsqrt(deg)[:, None]
    hs = jnp.dot(x_ref[0], w_ref[...], preferred_element_type=jnp.float32)
    hs_ref[0] = hs * dinv
    dinv_ref[0] = dinv


def _tc_k1(x, w1, degp):
    return pl.pallas_call(
        _k1_body,
        grid=(BATCH, N // BLK),
        in_specs=[
            pl.BlockSpec((1, BLK, D), lambda b, i: (b, i, 0)),
            pl.BlockSpec((D, D), lambda b, i: (0, 0)),
            pl.BlockSpec((1, BLK, NW), lambda b, i: (b, i, 0)),
        ],
        out_specs=[
            pl.BlockSpec((1, BLK, D), lambda b, i: (b, i, 0)),
            pl.BlockSpec((1, BLK, 1), lambda b, i: (b, i, 0)),
        ],
        out_shape=[
            jax.ShapeDtypeStruct((BATCH, N, D), jnp.float32),
            jax.ShapeDtypeStruct((BATCH, N, 1), jnp.float32),
        ],
    )(x, w1, degp)


def _k2_body(p_ref, hs_ref, dinv_ref, b1_ref, w2_ref, out_ref):
    dinv = dinv_ref[0]
    agg = p_ref[0, 0] + p_ref[0, 1] + hs_ref[0]
    h1 = dinv * agg + b1_ref[0]
    h1 = jnp.where(h1 > 0, h1, jnp.exp(jnp.minimum(h1, 0.0)) - 1.0)
    hs2 = jnp.dot(h1, w2_ref[...], preferred_element_type=jnp.float32)
    out_ref[0] = hs2 * dinv


def _tc_k2(p, hs, dinv, b1, w2):
    return pl.pallas_call(
        _k2_body,
        grid=(BATCH, N // BLK),
        in_specs=[
            pl.BlockSpec((1, NC, BLK, D), lambda b, i: (b, 0, i, 0)),
            pl.BlockSpec((1, BLK, D), lambda b, i: (b, i, 0)),
            pl.BlockSpec((1, BLK, 1), lambda b, i: (b, i, 0)),
            pl.BlockSpec((1, D), lambda b, i: (0, 0)),
            pl.BlockSpec((D, D), lambda b, i: (0, 0)),
        ],
        out_specs=pl.BlockSpec((1, BLK, D), lambda b, i: (b, i, 0)),
        out_shape=jax.ShapeDtypeStruct((BATCH, N, D), jnp.float32),
    )(p, hs, dinv, b1, w2)


def _k3_body(q_ref, hs_ref, dinv_ref, b2_ref, out_ref):
    dinv = dinv_ref[0]
    agg = q_ref[0, 0] + q_ref[0, 1] + hs_ref[0]
    out_ref[0] = dinv * agg + b2_ref[0]


def _tc_k3(q, hs, dinv, b2):
    return pl.pallas_call(
        _k3_body,
        grid=(BATCH, N // BLK),
        in_specs=[
            pl.BlockSpec((1, NC, BLK, D), lambda b, i: (b, 0, i, 0)),
            pl.BlockSpec((1, BLK, D), lambda b, i: (b, i, 0)),
            pl.BlockSpec((1, BLK, 1), lambda b, i: (b, i, 0)),
            pl.BlockSpec((1, D), lambda b, i: (0, 0)),
        ],
        out_specs=pl.BlockSpec((1, BLK, D), lambda b, i: (b, i, 0)),
        out_shape=jax.ShapeDtypeStruct((BATCH, N, D), jnp.float32),
    )(q, hs, dinv, b2)


@jax.jit
def kernel(x, edge_index, W1, b1, W2, b2):
    src = edge_index[:, 0, :]
    dst = edge_index[:, 1, :]
    src_shift = src + (jnp.arange(BATCH, dtype=jnp.int32) * N)[:, None]
    src_r = src_shift.reshape(BATCH * NW * EPT)
    dst_r = dst.reshape(BATCH * NW, NCHUNK, KCH)
    dst_pad = jnp.concatenate(
        [dst.reshape(BATCH * NW, EPT),
         jnp.full((BATCH * NW, EPAD - EPT), N, dtype=jnp.int32)], axis=1
    ).reshape(BATCH * NW, HROW, 128)

    degp = (_deg_kernel(dst_pad).reshape(BATCH, NW, EPAD)[:, :, :N]
            .transpose(0, 2, 1))

    hs1, dinv = _tc_k1(x, W1, degp)

    p = _edge_kernel(hs1.reshape(BATCH * N, D), src_r, dst_r)
    p = p.reshape(BATCH, NC, N, D)

    hs2 = _tc_k2(p, hs1, dinv, b1.reshape(1, D), W2)

    q = _edge_kernel(hs2.reshape(BATCH * N, D), src_r, dst_r)
    q = q.reshape(BATCH, NC, N, D)

    return _tc_k3(q, hs2, dinv, b2.reshape(1, D))
